# Initial kernel scaffold; baseline (speedup 1.0000x reference)
import functools
import jax
import jax.numpy as jnp
from jax import lax
from jax.experimental import pallas as pl
from jax.experimental.pallas import tpu as pltpu

N = 10000
E = 320000
H = 128
HEADS = 4
BLK = 1000
GRID = N // BLK


def _mm_kernel(x_ref, w_ref, b_ref, o_ref, *, act):
    r = jnp.dot(x_ref[...], w_ref[...], preferred_element_type=jnp.float32)
    r = r + b_ref[...]
    o_ref[...] = act(r)


def _mm(x, w, b, act=lambda v: v):
    dout = w.shape[1]
    return pl.pallas_call(
        functools.partial(_mm_kernel, act=act),
        grid=(GRID,),
        in_specs=[
            pl.BlockSpec((BLK, x.shape[1]), lambda i: (i, 0)),
            pl.BlockSpec((w.shape[0], dout), lambda i: (0, 0)),
            pl.BlockSpec((1, dout), lambda i: (0, 0)),
        ],
        out_specs=pl.BlockSpec((BLK, dout), lambda i: (i, 0)),
        out_shape=jax.ShapeDtypeStruct((N, dout), jnp.float32),
    )(x, w, b.reshape(1, dout))


def _mm_scale_kernel(x_ref, w_ref, s_ref, o_ref):
    r = jnp.dot(x_ref[...], w_ref[...], preferred_element_type=jnp.float32)
    o_ref[...] = r * s_ref[...]


def _mm_scale(x, w, s):
    dout = w.shape[1]
    return pl.pallas_call(
        _mm_scale_kernel,
        grid=(GRID,),
        in_specs=[
            pl.BlockSpec((BLK, x.shape[1]), lambda i: (i, 0)),
            pl.BlockSpec((w.shape[0], dout), lambda i: (0, 0)),
            pl.BlockSpec((BLK, 1), lambda i: (i, 0)),
        ],
        out_specs=pl.BlockSpec((BLK, dout), lambda i: (i, 0)),
        out_shape=jax.ShapeDtypeStruct((N, dout), jnp.float32),
    )(x, w, s.reshape(N, 1))


def _gcn_post_kernel(agg_ref, y_ref, s_ref, b_ref, hp_ref, o_ref, *, relu_res):
    r = (agg_ref[...] + y_ref[...]) * s_ref[...] + b_ref[...]
    if relu_res:
        r = jnp.maximum(r, 0.0) + hp_ref[...]
    o_ref[...] = r


def _gcn_post(agg, y, dinv, b, h_prev, relu_res):
    return pl.pallas_call(
        functools.partial(_gcn_post_kernel, relu_res=relu_res),
        grid=(GRID,),
        in_specs=[
            pl.BlockSpec((BLK, H), lambda i: (i, 0)),
            pl.BlockSpec((BLK, H), lambda i: (i, 0)),
            pl.BlockSpec((BLK, 1), lambda i: (i, 0)),
            pl.BlockSpec((1, H), lambda i: (0, 0)),
            pl.BlockSpec((BLK, H), lambda i: (i, 0)),
        ],
        out_specs=pl.BlockSpec((BLK, H), lambda i: (i, 0)),
        out_shape=jax.ShapeDtypeStruct((N, H), jnp.float32),
    )(agg, y, dinv.reshape(N, 1), b.reshape(1, H), h_prev)


def _gat_dense_kernel(x_ref, w_ref, as_ref, ad_ref, xw_ref, asn_ref, adn_ref):
    xw = jnp.dot(x_ref[...], w_ref[...], preferred_element_type=jnp.float32)
    xw_ref[...] = xw
    asn_ref[...] = jnp.dot(xw, as_ref[...], preferred_element_type=jnp.float32)
    adn_ref[...] = jnp.dot(xw, ad_ref[...], preferred_element_type=jnp.float32)


def _gat_dense(h3, W_gat, As, Ad):
    return pl.pallas_call(
        _gat_dense_kernel,
        grid=(GRID,),
        in_specs=[
            pl.BlockSpec((BLK, H), lambda i: (i, 0)),
            pl.BlockSpec((H, HEADS * H), lambda i: (0, 0)),
            pl.BlockSpec((HEADS * H, HEADS), lambda i: (0, 0)),
            pl.BlockSpec((HEADS * H, HEADS), lambda i: (0, 0)),
        ],
        out_specs=[
            pl.BlockSpec((BLK, HEADS * H), lambda i: (i, 0)),
            pl.BlockSpec((BLK, HEADS), lambda i: (i, 0)),
            pl.BlockSpec((BLK, HEADS), lambda i: (i, 0)),
        ],
        out_shape=[
            jax.ShapeDtypeStruct((N, HEADS * H), jnp.float32),
            jax.ShapeDtypeStruct((N, HEADS), jnp.float32),
            jax.ShapeDtypeStruct((N, HEADS), jnp.float32),
        ],
    )(h3, W_gat, As, Ad)


def _gat_final_kernel(h3_ref, acc_ref, st_ref, b_ref, wo_ref, bo_ref, o_ref):
    h_att = (acc_ref[...] + st_ref[...]) * (1.0 / HEADS) + b_ref[...]
    hf = h3_ref[...] + h_att
    o_ref[...] = jnp.dot(hf, wo_ref[...], preferred_element_type=jnp.float32) + bo_ref[...]


def _gat_final(h3, acc, self_term, b_gat, W_out, b_out):
    return pl.pallas_call(
        _gat_final_kernel,
        grid=(GRID,),
        in_specs=[
            pl.BlockSpec((BLK, H), lambda i: (i, 0)),
            pl.BlockSpec((BLK, H), lambda i: (i, 0)),
            pl.BlockSpec((BLK, H), lambda i: (i, 0)),
            pl.BlockSpec((1, H), lambda i: (0, 0)),
            pl.BlockSpec((H, 1), lambda i: (0, 0)),
            pl.BlockSpec((1, 1), lambda i: (0, 0)),
        ],
        out_specs=pl.BlockSpec((BLK, 1), lambda i: (i, 0)),
        out_shape=jax.ShapeDtypeStruct((N, 1), jnp.float32),
    )(h3, acc, self_term, b_gat.reshape(1, H), W_out, b_out.reshape(1, 1))


def kernel(x, edge_index, edge_weight, W_in, b_in, W_g1, b_g1, W_g2, b_g2,
           W_g3, b_g3, W_gat, att_src, att_dst, b_gat, W_out, b_out):
    src = edge_index[0]
    dst = edge_index[1]
    ew = edge_weight

    deg = jax.ops.segment_sum(ew, dst, num_segments=N) + 1.0
    dinv = lax.rsqrt(deg)

    h0 = _mm(x, W_in, b_in, act=lambda v: jnp.maximum(v, 0.0))

    def gcn(h, W, b, h_prev, relu_res):
        y = _mm_scale(h, W, dinv)
        agg = jax.ops.segment_sum(ew[:, None] * y[src], dst, num_segments=N)
        return _gcn_post(agg, y, dinv, b, h_prev, relu_res)

    h1 = gcn(h0, W_g1, b_g1, h0, True)
    h2 = gcn(h1, W_g2, b_g2, h1, True)
    h3 = gcn(h2, W_g3, b_g3, h2, False)

    As = (jnp.eye(HEADS, dtype=jnp.float32)[:, None, :]
          * att_src[:, :, None]).reshape(HEADS * H, HEADS)
    Ad = (jnp.eye(HEADS, dtype=jnp.float32)[:, None, :]
          * att_dst[:, :, None]).reshape(HEADS * H, HEADS)
    xwg, asn, adn = _gat_dense(h3, W_gat, As, Ad)

    Cg = jax.nn.leaky_relu(jnp.max(asn, axis=0) + jnp.max(adn, axis=0), 0.2)
    t = asn[src] + adn[dst]
    u = jnp.exp(jax.nn.leaky_relu(t, 0.2) - Cg[None, :])
    u_self = jnp.exp(jax.nn.leaky_relu(asn + adn, 0.2) - Cg[None, :])
    denom = jax.ops.segment_sum(u, dst, num_segments=N) + u_self
    invd = 1.0 / denom
    att = u * invd[dst]
    xwr = xwg.reshape(N, HEADS, H)
    combo = jnp.einsum('eh,ehk->ek', att, xwr[src])
    acc = jax.ops.segment_sum(combo, dst, num_segments=N)
    self_term = jnp.einsum('nh,nhk->nk', u_self * invd, xwr)

    return _gat_final(h3, acc, self_term, b_gat, W_out, b_out)


# speedup vs baseline: 13.6470x; 13.6470x over previous
import functools
import jax
import jax.numpy as jnp
from jax import lax
from jax.experimental import pallas as pl
from jax.experimental.pallas import tpu as pltpu
from jax.experimental.pallas import tpu_sc as plsc

N = 10000
E = 320000
H = 128
HEADS = 4
BLK = 1000
GRID = N // BLK

NC = 2
NS = 16
NW = NC * NS
EPT = E // NW
CH = 80
NCHUNK = EPT // CH
RPT = 624
ZROWS = 80
TAIL0 = N - NS * RPT
CP = pltpu.CompilerParams(needs_layout_passes=False)


def _mesh():
    return plsc.VectorSubcoreMesh(core_axis_name="c", subcore_axis_name="s",
                                  num_cores=NC, num_subcores=NS)


def _zero_buf(buf, nrows, width):
    def zrow(i, _):
        for j in range(width // 16):
            buf[i, 16 * j:16 * (j + 1)] = jnp.zeros((16,), jnp.float32)
        return 0
    lax.fori_loop(0, nrows, zrow, 0)


def _zero_acc(s, zbuf, acc, rpc):
    n_full, rem = RPT // rpc, RPT % rpc

    def zcopy(k, _):
        r0 = pl.multiple_of(s * RPT + k * rpc, 8)
        pltpu.sync_copy(zbuf.at[pl.ds(0, rpc), :], acc.at[pl.ds(r0, rpc), :])
        return 0
    lax.fori_loop(0, n_full, zcopy, 0)
    if rem:
        rr = pl.multiple_of(s * RPT + n_full * rpc, 8)
        pltpu.sync_copy(zbuf.at[pl.ds(0, rem), :], acc.at[pl.ds(rr, rem), :])

    @pl.when(s == 0)
    def _():
        pltpu.sync_copy(zbuf.at[pl.ds(0, TAIL0), :],
                        acc.at[pl.ds(NS * RPT, TAIL0), :])


def _write_acc(c, s, acc, out_hbm, rpc):
    n_full, rem = RPT // rpc, RPT % rpc

    def wcopy(k, _):
        r0 = pl.multiple_of(s * RPT + k * rpc, 8)
        pltpu.sync_copy(acc.at[pl.ds(r0, rpc), :],
                        out_hbm.at[c, pl.ds(r0, rpc), :])
        return 0
    lax.fori_loop(0, n_full, wcopy, 0)
    if rem:
        rr = pl.multiple_of(s * RPT + n_full * rpc, 8)
        pltpu.sync_copy(acc.at[pl.ds(rr, rem), :],
                        out_hbm.at[c, pl.ds(rr, rem), :])

    @pl.when(s == 0)
    def _():
        pltpu.sync_copy(acc.at[pl.ds(NS * RPT, TAIL0), :],
                        out_hbm.at[c, pl.ds(NS * RPT, TAIL0), :])


def _sc_agg_body(y_hbm, src_hbm, dst_hbm, ew_hbm, out_hbm,
                 srcb, dstb, ewb, rows, zbuf, acc, sem):
    c = lax.axis_index("c")
    s = lax.axis_index("s")
    base = (c * NS + s) * EPT

    _zero_buf(zbuf, ZROWS, H)
    _zero_acc(s, zbuf, acc, ZROWS)
    plsc.subcore_barrier()

    def chunk(ci, _):
        b = pl.multiple_of(base + ci * CH, 8)
        pltpu.sync_copy(src_hbm.at[pl.ds(b, CH)], srcb)
        pltpu.sync_copy(dst_hbm.at[pl.ds(b, CH)], dstb)
        pltpu.sync_copy(ew_hbm.at[pl.ds(b, CH)], ewb)
        pltpu.async_copy(y_hbm.at[srcb], rows, sem).wait()

        def edge(i, _):
            w = plsc.load_gather(ewb, [jnp.full((16,), i, jnp.int32)])
            for j in range(H // 16):
                sl = pl.ds(16 * j, 16)
                rows[i, sl] = rows[i, sl] * w
            return 0
        lax.fori_loop(0, CH, edge, 0)
        pltpu.sync_copy(rows, acc.at[dstb], add=True)
        return 0
    lax.fori_loop(0, NCHUNK, chunk, 0)

    plsc.subcore_barrier()
    _write_acc(c, s, acc, out_hbm, ZROWS)


def _sc_agg(y, src, dst, ew):
    fn = pl.kernel(
        _sc_agg_body,
        out_type=jax.ShapeDtypeStruct((NC, N, H), jnp.float32),
        mesh=_mesh(), compiler_params=CP,
        scratch_types=[
            pltpu.VMEM((CH,), jnp.int32),
            pltpu.VMEM((CH,), jnp.int32),
            pltpu.VMEM((CH,), jnp.float32),
            pltpu.VMEM((CH, H), jnp.float32),
            pltpu.VMEM((ZROWS, H), jnp.float32),
            pltpu.VMEM_SHARED((N, H), jnp.float32),
            pltpu.SemaphoreType.DMA,
        ],
    )
    return fn(y, src, dst, ew)


def _sc_gat_u_body(asn_hbm, adn_hbm, src_hbm, dst_hbm, cg_hbm, u_hbm,
                   asn_t, adn_t, cgb, srcb, dstb, ub):
    c = lax.axis_index("c")
    s = lax.axis_index("s")
    base = (c * NS + s) * EPT

    pltpu.sync_copy(asn_hbm, asn_t)
    pltpu.sync_copy(adn_hbm, adn_t)
    pltpu.sync_copy(cg_hbm, cgb)

    lane = jnp.arange(16, dtype=jnp.int32)
    eoff = lane // 4
    hoff = lane - eoff * 4
    cgv = cgb[...]

    def chunk(ci, _):
        b = pl.multiple_of(base + ci * CHA, 8)
        pltpu.sync_copy(src_hbm.at[pl.ds(b, CHA)], srcb)
        pltpu.sync_copy(dst_hbm.at[pl.ds(b, CHA)], dstb)

        def group(g, _):
            eidx = g * 4 + eoff
            srcv = plsc.load_gather(srcb, [eidx])
            dstv = plsc.load_gather(dstb, [eidx])
            av = plsc.load_gather(asn_t, [srcv * 4 + hoff])
            dv = plsc.load_gather(adn_t, [dstv * 4 + hoff])
            tt = av + dv
            uu = jnp.exp(jnp.maximum(tt, 0.2 * tt) - cgv)
            ub[pl.ds(pl.multiple_of(g * 16, 8), 16)] = uu
            return 0
        lax.fori_loop(0, CHA // 4, group, 0)
        b4 = pl.multiple_of((base + ci * CHA) * 4, 8)
        pltpu.sync_copy(ub, u_hbm.at[pl.ds(b4, CHA * 4)])
        return 0
    lax.fori_loop(0, EPT // CHA, chunk, 0)


def _sc_gat_u(asn, adn, src, dst, cg):
    fn = pl.kernel(
        _sc_gat_u_body,
        out_type=jax.ShapeDtypeStruct((E * 4,), jnp.float32),
        mesh=_mesh(), compiler_params=CP,
        scratch_types=[
            pltpu.VMEM((N * 4,), jnp.float32),
            pltpu.VMEM((N * 4,), jnp.float32),
            pltpu.VMEM((16,), jnp.float32),
            pltpu.VMEM((CHA,), jnp.int32),
            pltpu.VMEM((CHA,), jnp.int32),
            pltpu.VMEM((CHA * 4,), jnp.float32),
        ],
    )
    return fn(asn, adn, src, dst, cg)


def _sc_dsum_body(u_hbm, dst_hbm, out_hbm, dstb, ub, rows, acc):
    c = lax.axis_index("c")
    s = lax.axis_index("s")
    base = (c * NS + s) * EPT

    _zero_buf(rows, CH, H)
    _zero_acc(s, rows, acc, CH)
    plsc.subcore_barrier()

    lane = jnp.arange(16, dtype=jnp.int32)
    lane4 = jnp.minimum(lane, 3)
    mask4 = (lane < 4).astype(jnp.float32)

    def chunk(ci, _):
        b = pl.multiple_of(base + ci * CH, 8)
        pltpu.sync_copy(dst_hbm.at[pl.ds(b, CH)], dstb)
        b4 = pl.multiple_of((base + ci * CH) * 4, 8)
        pltpu.sync_copy(u_hbm.at[pl.ds(b4, CH * 4)], ub)

        def edge(i, _):
            u16 = plsc.load_gather(ub, [i * 4 + lane4])
            rows[i, 0:16] = u16 * mask4
            return 0
        lax.fori_loop(0, CH, edge, 0)
        pltpu.sync_copy(rows, acc.at[dstb], add=True)
        return 0
    lax.fori_loop(0, NCHUNK, chunk, 0)

    plsc.subcore_barrier()
    _write_acc(c, s, acc, out_hbm, CH)


def _sc_dsum(u, dst):
    fn = pl.kernel(
        _sc_dsum_body,
        out_type=jax.ShapeDtypeStruct((NC, N, H), jnp.float32),
        mesh=_mesh(), compiler_params=CP,
        scratch_types=[
            pltpu.VMEM((CH,), jnp.int32),
            pltpu.VMEM((CH * 4,), jnp.float32),
            pltpu.VMEM((CH, H), jnp.float32),
            pltpu.VMEM_SHARED((N, H), jnp.float32),
        ],
    )
    return fn(u, dst)


CHA = 400


def _sc_att_body(u_hbm, dst_hbm, invd_hbm, att_hbm, invd_t, dstb, ub):
    c = lax.axis_index("c")
    s = lax.axis_index("s")
    base = (c * NS + s) * EPT

    pltpu.sync_copy(invd_hbm, invd_t)
    lane = jnp.arange(16, dtype=jnp.int32)
    eoff = lane // 4
    hoff = lane - eoff * 4

    def chunk(ci, _):
        b = pl.multiple_of(base + ci * CHA, 8)
        pltpu.sync_copy(dst_hbm.at[pl.ds(b, CHA)], dstb)
        b4 = pl.multiple_of((base + ci * CHA) * 4, 8)
        pltpu.sync_copy(u_hbm.at[pl.ds(b4, CHA * 4)], ub)

        def group(g, _):
            eidx = g * 4 + eoff
            dstv = plsc.load_gather(dstb, [eidx])
            iv = plsc.load_gather(invd_t, [dstv * 4 + hoff])
            off = pl.multiple_of(g * 16, 8)
            ub[pl.ds(off, 16)] = ub[pl.ds(off, 16)] * iv
            return 0
        lax.fori_loop(0, CHA // 4, group, 0)
        pltpu.sync_copy(ub, att_hbm.at[pl.ds(b4, CHA * 4)])
        return 0
    lax.fori_loop(0, EPT // CHA, chunk, 0)


def _sc_att(u, dst, invd):
    fn = pl.kernel(
        _sc_att_body,
        out_type=jax.ShapeDtypeStruct((E * 4,), jnp.float32),
        mesh=_mesh(), compiler_params=CP,
        scratch_types=[
            pltpu.VMEM((N * 4,), jnp.float32),
            pltpu.VMEM((CHA,), jnp.int32),
            pltpu.VMEM((CHA * 4,), jnp.float32),
        ],
    )
    return fn(u, dst, invd)


CHV = 40


def _sc_gat_value_body(xwg_hbm, src_hbm, dst_hbm, att_hbm, out_hbm,
                       srcb, dstb, attb, rows, combo, acc, sem):
    c = lax.axis_index("c")
    s = lax.axis_index("s")
    base = (c * NS + s) * EPT

    _zero_buf(combo, CHV, H)
    _zero_acc(s, combo, acc, CHV)
    plsc.subcore_barrier()

    def chunk(ci, _):
        b = pl.multiple_of(base + ci * CHV, 8)
        pltpu.sync_copy(src_hbm.at[pl.ds(b, CHV)], srcb)
        pltpu.sync_copy(dst_hbm.at[pl.ds(b, CHV)], dstb)
        b4 = pl.multiple_of((base + ci * CHV) * 4, 8)
        pltpu.sync_copy(att_hbm.at[pl.ds(b4, CHV * 4)], attb)
        pltpu.async_copy(xwg_hbm.at[srcb], rows, sem).wait()

        def edge(i, _):
            a = [plsc.load_gather(attb, [jnp.full((16,), i * 4 + h, jnp.int32)])
                 for h in range(HEADS)]
            for j in range(H // 16):
                acc16 = a[0] * rows[i, pl.ds(16 * j, 16)]
                for h in range(1, HEADS):
                    acc16 = acc16 + a[h] * rows[i, pl.ds(h * H + 16 * j, 16)]
                combo[i, pl.ds(16 * j, 16)] = acc16
            return 0
        lax.fori_loop(0, CHV, edge, 0)
        pltpu.sync_copy(combo, acc.at[dstb], add=True)
        return 0
    lax.fori_loop(0, EPT // CHV, chunk, 0)

    plsc.subcore_barrier()
    _write_acc(c, s, acc, out_hbm, CHV)


def _sc_gat_value(xwg, src, dst, att):
    fn = pl.kernel(
        _sc_gat_value_body,
        out_type=jax.ShapeDtypeStruct((NC, N, H), jnp.float32),
        mesh=_mesh(), compiler_params=CP,
        scratch_types=[
            pltpu.VMEM((CHV,), jnp.int32),
            pltpu.VMEM((CHV,), jnp.int32),
            pltpu.VMEM((CHV * 4,), jnp.float32),
            pltpu.VMEM((CHV, HEADS * H), jnp.float32),
            pltpu.VMEM((CHV, H), jnp.float32),
            pltpu.VMEM_SHARED((N, H), jnp.float32),
            pltpu.SemaphoreType.DMA,
        ],
    )
    return fn(xwg, src, dst, att)


def _mm_kernel(x_ref, w_ref, b_ref, o_ref, *, act):
    r = jnp.dot(x_ref[...], w_ref[...], preferred_element_type=jnp.float32)
    r = r + b_ref[...]
    o_ref[...] = act(r)


def _mm(x, w, b, act=lambda v: v):
    dout = w.shape[1]
    return pl.pallas_call(
        functools.partial(_mm_kernel, act=act),
        grid=(GRID,),
        in_specs=[
            pl.BlockSpec((BLK, x.shape[1]), lambda i: (i, 0)),
            pl.BlockSpec((w.shape[0], dout), lambda i: (0, 0)),
            pl.BlockSpec((1, dout), lambda i: (0, 0)),
        ],
        out_specs=pl.BlockSpec((BLK, dout), lambda i: (i, 0)),
        out_shape=jax.ShapeDtypeStruct((N, dout), jnp.float32),
    )(x, w, b.reshape(1, dout))


def _mm_scale_kernel(x_ref, w_ref, s_ref, o_ref):
    r = jnp.dot(x_ref[...], w_ref[...], preferred_element_type=jnp.float32)
    o_ref[...] = r * s_ref[...]


def _mm_scale(x, w, s):
    dout = w.shape[1]
    return pl.pallas_call(
        _mm_scale_kernel,
        grid=(GRID,),
        in_specs=[
            pl.BlockSpec((BLK, x.shape[1]), lambda i: (i, 0)),
            pl.BlockSpec((w.shape[0], dout), lambda i: (0, 0)),
            pl.BlockSpec((BLK, 1), lambda i: (i, 0)),
        ],
        out_specs=pl.BlockSpec((BLK, dout), lambda i: (i, 0)),
        out_shape=jax.ShapeDtypeStruct((N, dout), jnp.float32),
    )(x, w, s.reshape(N, 1))


def _gcn_post_kernel(a0_ref, a1_ref, y_ref, s_ref, b_ref, hp_ref, o_ref,
                     *, relu_res):
    r = (a0_ref[0] + a1_ref[0] + y_ref[...]) * s_ref[...] + b_ref[...]
    if relu_res:
        r = jnp.maximum(r, 0.0) + hp_ref[...]
    o_ref[...] = r


def _gcn_post(parts, y, dinv, b, h_prev, relu_res):
    return pl.pallas_call(
        functools.partial(_gcn_post_kernel, relu_res=relu_res),
        grid=(GRID,),
        in_specs=[
            pl.BlockSpec((1, BLK, H), lambda i: (0, i, 0)),
            pl.BlockSpec((1, BLK, H), lambda i: (1, i, 0)),
            pl.BlockSpec((BLK, H), lambda i: (i, 0)),
            pl.BlockSpec((BLK, 1), lambda i: (i, 0)),
            pl.BlockSpec((1, H), lambda i: (0, 0)),
            pl.BlockSpec((BLK, H), lambda i: (i, 0)),
        ],
        out_specs=pl.BlockSpec((BLK, H), lambda i: (i, 0)),
        out_shape=jax.ShapeDtypeStruct((N, H), jnp.float32),
    )(parts, parts, y, dinv.reshape(N, 1), b.reshape(1, H), h_prev)


def _gat_dense_kernel(x_ref, w_ref, as_ref, ad_ref, xw_ref, asn_ref, adn_ref):
    xw = jnp.dot(x_ref[...], w_ref[...], preferred_element_type=jnp.float32)
    xw_ref[...] = xw
    asn_ref[...] = jnp.dot(xw, as_ref[...], preferred_element_type=jnp.float32)
    adn_ref[...] = jnp.dot(xw, ad_ref[...], preferred_element_type=jnp.float32)


def _gat_dense(h3, W_gat, As, Ad):
    return pl.pallas_call(
        _gat_dense_kernel,
        grid=(GRID,),
        in_specs=[
            pl.BlockSpec((BLK, H), lambda i: (i, 0)),
            pl.BlockSpec((H, HEADS * H), lambda i: (0, 0)),
            pl.BlockSpec((HEADS * H, HEADS), lambda i: (0, 0)),
            pl.BlockSpec((HEADS * H, HEADS), lambda i: (0, 0)),
        ],
        out_specs=[
            pl.BlockSpec((BLK, HEADS * H), lambda i: (i, 0)),
            pl.BlockSpec((BLK, HEADS), lambda i: (i, 0)),
            pl.BlockSpec((BLK, HEADS), lambda i: (i, 0)),
        ],
        out_shape=[
            jax.ShapeDtypeStruct((N, HEADS * H), jnp.float32),
            jax.ShapeDtypeStruct((N, HEADS), jnp.float32),
            jax.ShapeDtypeStruct((N, HEADS), jnp.float32),
        ],
    )(h3, W_gat, As, Ad)


def _gat_mid_kernel(d0_ref, d1_ref, asn_ref, adn_ref, cg_ref, xw_ref,
                    invd_ref, st_ref):
    t = asn_ref[...] + adn_ref[...]
    u_self = jnp.exp(jnp.maximum(t, 0.2 * t) - cg_ref[...])
    den = d0_ref[0][:, :HEADS] + d1_ref[0][:, :HEADS] + u_self
    invd = 1.0 / den
    invd_ref[...] = invd
    w = u_self * invd
    xw = xw_ref[...]
    st = w[:, 0:1] * xw[:, 0:H]
    for h in range(1, HEADS):
        st = st + w[:, h:h + 1] * xw[:, h * H:(h + 1) * H]
    st_ref[...] = st


def _gat_mid(dparts, asn, adn, cg, xwg):
    return pl.pallas_call(
        _gat_mid_kernel,
        grid=(GRID,),
        in_specs=[
            pl.BlockSpec((1, BLK, H), lambda i: (0, i, 0)),
            pl.BlockSpec((1, BLK, H), lambda i: (1, i, 0)),
            pl.BlockSpec((BLK, HEADS), lambda i: (i, 0)),
            pl.BlockSpec((BLK, HEADS), lambda i: (i, 0)),
            pl.BlockSpec((1, HEADS), lambda i: (0, 0)),
            pl.BlockSpec((BLK, HEADS * H), lambda i: (i, 0)),
        ],
        out_specs=[
            pl.BlockSpec((BLK, HEADS), lambda i: (i, 0)),
            pl.BlockSpec((BLK, H), lambda i: (i, 0)),
        ],
        out_shape=[
            jax.ShapeDtypeStruct((N, HEADS), jnp.float32),
            jax.ShapeDtypeStruct((N, H), jnp.float32),
        ],
    )(dparts, dparts, asn, adn, cg.reshape(1, HEADS), xwg)


def _gat_final_kernel(h3_ref, p0_ref, p1_ref, st_ref, b_ref, wo_ref, bo_ref,
                      o_ref):
    h_att = ((p0_ref[0] + p1_ref[0] + st_ref[...]) * (1.0 / HEADS)
             + b_ref[...])
    hf = h3_ref[...] + h_att
    o_ref[...] = (jnp.dot(hf, wo_ref[...], preferred_element_type=jnp.float32)
                  + bo_ref[...])


def _gat_final(h3, vparts, st, b_gat, W_out, b_out):
    return pl.pallas_call(
        _gat_final_kernel,
        grid=(GRID,),
        in_specs=[
            pl.BlockSpec((BLK, H), lambda i: (i, 0)),
            pl.BlockSpec((1, BLK, H), lambda i: (0, i, 0)),
            pl.BlockSpec((1, BLK, H), lambda i: (1, i, 0)),
            pl.BlockSpec((BLK, H), lambda i: (i, 0)),
            pl.BlockSpec((1, H), lambda i: (0, 0)),
            pl.BlockSpec((H, 1), lambda i: (0, 0)),
            pl.BlockSpec((1, 1), lambda i: (0, 0)),
        ],
        out_specs=pl.BlockSpec((BLK, 1), lambda i: (i, 0)),
        out_shape=jax.ShapeDtypeStruct((N, 1), jnp.float32),
    )(h3, vparts, vparts, st, b_gat.reshape(1, H), W_out,
      b_out.reshape(1, 1))


def kernel(x, edge_index, edge_weight, W_in, b_in, W_g1, b_g1, W_g2, b_g2,
           W_g3, b_g3, W_gat, att_src, att_dst, b_gat, W_out, b_out):
    src = edge_index[0]
    dst = edge_index[1]
    ew = edge_weight

    ones = jnp.ones((N, H), jnp.float32)
    deg_parts = _sc_agg(ones, src, dst, ew)
    deg = deg_parts[0, :, 0] + deg_parts[1, :, 0] + 1.0
    dinv = lax.rsqrt(deg)

    h0 = _mm(x, W_in, b_in, act=lambda v: jnp.maximum(v, 0.0))

    def gcn(h, W, b, h_prev, relu_res):
        y = _mm_scale(h, W, dinv)
        parts = _sc_agg(y, src, dst, ew)
        return _gcn_post(parts, y, dinv, b, h_prev, relu_res)

    h1 = gcn(h0, W_g1, b_g1, h0, True)
    h2 = gcn(h1, W_g2, b_g2, h1, True)
    h3 = gcn(h2, W_g3, b_g3, h2, False)

    As = (jnp.eye(HEADS, dtype=jnp.float32)[:, None, :]
          * att_src[:, :, None]).reshape(HEADS * H, HEADS)
    Ad = (jnp.eye(HEADS, dtype=jnp.float32)[:, None, :]
          * att_dst[:, :, None]).reshape(HEADS * H, HEADS)
    xwg, asn, adn = _gat_dense(h3, W_gat, As, Ad)

    cg = jax.nn.leaky_relu(jnp.max(asn, axis=0) + jnp.max(adn, axis=0), 0.2)
    cg16 = jnp.tile(cg, 4)

    u = _sc_gat_u(asn.reshape(N * HEADS), adn.reshape(N * HEADS),
                  src, dst, cg16)
    dparts = _sc_dsum(u, dst)
    invd, st = _gat_mid(dparts, asn, adn, cg, xwg)
    att = _sc_att(u, dst, invd.reshape(N * HEADS))
    vparts = _sc_gat_value(xwg, src, dst, att)

    return _gat_final(h3, vparts, st, b_gat, W_out, b_out)


# speedup vs baseline: 15.2333x; 1.1162x over previous
import functools
import jax
import jax.numpy as jnp
from jax import lax
from jax.experimental import pallas as pl
from jax.experimental.pallas import tpu as pltpu
from jax.experimental.pallas import tpu_sc as plsc

N = 10000
E = 320000
H = 128
HEADS = 4
BLK = 1000
GRID = N // BLK

NC = 2
NS = 16
NW = NC * NS
EPT = E // NW
CH = 80
NCHUNK = EPT // CH
RPT = 624
ZROWS = 80
TAIL0 = N - NS * RPT
CP = pltpu.CompilerParams(needs_layout_passes=False)


def _mesh():
    return plsc.VectorSubcoreMesh(core_axis_name="c", subcore_axis_name="s",
                                  num_cores=NC, num_subcores=NS)


def _zero_buf(buf, nrows, width):
    def zrow(i, _):
        for j in range(width // 16):
            buf[i, 16 * j:16 * (j + 1)] = jnp.zeros((16,), jnp.float32)
        return 0
    lax.fori_loop(0, nrows, zrow, 0)


def _zero_acc(s, zbuf, acc, rpc):
    n_full, rem = RPT // rpc, RPT % rpc

    def zcopy(k, _):
        r0 = pl.multiple_of(s * RPT + k * rpc, 8)
        pltpu.sync_copy(zbuf.at[pl.ds(0, rpc), :], acc.at[pl.ds(r0, rpc), :])
        return 0
    lax.fori_loop(0, n_full, zcopy, 0)
    if rem:
        rr = pl.multiple_of(s * RPT + n_full * rpc, 8)
        pltpu.sync_copy(zbuf.at[pl.ds(0, rem), :], acc.at[pl.ds(rr, rem), :])

    @pl.when(s == 0)
    def _():
        pltpu.sync_copy(zbuf.at[pl.ds(0, TAIL0), :],
                        acc.at[pl.ds(NS * RPT, TAIL0), :])


def _write_acc(c, s, acc, out_hbm, rpc):
    n_full, rem = RPT // rpc, RPT % rpc

    def wcopy(k, _):
        r0 = pl.multiple_of(s * RPT + k * rpc, 8)
        pltpu.sync_copy(acc.at[pl.ds(r0, rpc), :],
                        out_hbm.at[c, pl.ds(r0, rpc), :])
        return 0
    lax.fori_loop(0, n_full, wcopy, 0)
    if rem:
        rr = pl.multiple_of(s * RPT + n_full * rpc, 8)
        pltpu.sync_copy(acc.at[pl.ds(rr, rem), :],
                        out_hbm.at[c, pl.ds(rr, rem), :])

    @pl.when(s == 0)
    def _():
        pltpu.sync_copy(acc.at[pl.ds(NS * RPT, TAIL0), :],
                        out_hbm.at[c, pl.ds(NS * RPT, TAIL0), :])


def _sc_agg_body(y_hbm, src3_hbm, dst3_hbm, ew2_hbm, out_hbm,
                 srcb2, dstb2, ewb, rows, acc, sem):
    c = lax.axis_index("c")
    s = lax.axis_index("s")
    wid = c * NS + s

    _zero_buf(rows, CH, H)
    _zero_acc(s, rows, acc, CH)
    pltpu.sync_copy(src3_hbm.at[wid], srcb2)
    pltpu.sync_copy(dst3_hbm.at[wid], dstb2)
    plsc.subcore_barrier()

    def chunk(ci, _):
        b = pl.multiple_of(wid * EPT + ci * CH, 8)
        pltpu.sync_copy(ew2_hbm.at[pl.ds(b, CH)], ewb)
        pltpu.async_copy(y_hbm.at[srcb2.at[ci]], rows, sem).wait()

        def edge(i, _):
            w = plsc.load_gather(ewb, [jnp.full((16,), i, jnp.int32)])
            for j in range(H // 16):
                sl = pl.ds(16 * j, 16)
                rows[i, sl] = rows[i, sl] * w
            return 0
        lax.fori_loop(0, CH, edge, 0)
        pltpu.sync_copy(rows, acc.at[dstb2.at[ci]], add=True)
        return 0
    lax.fori_loop(0, NCHUNK, chunk, 0)

    plsc.subcore_barrier()
    _write_acc(c, s, acc, out_hbm, CH)


def _sc_agg(y, src, dst, ew):
    fn = pl.kernel(
        _sc_agg_body,
        out_type=jax.ShapeDtypeStruct((NC, N, H), jnp.float32),
        mesh=_mesh(), compiler_params=CP,
        scratch_types=[
            pltpu.VMEM((NCHUNK, CH), jnp.int32),
            pltpu.VMEM((NCHUNK, CH), jnp.int32),
            pltpu.VMEM((CH,), jnp.float32),
            pltpu.VMEM((CH, H), jnp.float32),
            pltpu.VMEM_SHARED((N, H), jnp.float32),
            pltpu.SemaphoreType.DMA,
        ],
    )
    return fn(y, src.reshape(NW, NCHUNK, CH), dst.reshape(NW, NCHUNK, CH),
              ew)


def _sc_gat_u_body(asn_hbm, adn_hbm, src_hbm, dst_hbm, cg_hbm, u_hbm,
                   asn_t, adn_t, cgb, srcb, dstb, ub):
    c = lax.axis_index("c")
    s = lax.axis_index("s")
    base = (c * NS + s) * EPT

    pltpu.sync_copy(asn_hbm, asn_t)
    pltpu.sync_copy(adn_hbm, adn_t)
    pltpu.sync_copy(cg_hbm, cgb)

    lane = jnp.arange(16, dtype=jnp.int32)
    eoff = lane // 4
    hoff = lane - eoff * 4
    cgv = cgb[...]

    def chunk(ci, _):
        b = pl.multiple_of(base + ci * CHA, 8)
        pltpu.sync_copy(src_hbm.at[pl.ds(b, CHA)], srcb)
        pltpu.sync_copy(dst_hbm.at[pl.ds(b, CHA)], dstb)

        def group(g, _):
            eidx = g * 4 + eoff
            srcv = plsc.load_gather(srcb, [eidx])
            dstv = plsc.load_gather(dstb, [eidx])
            av = plsc.load_gather(asn_t, [srcv * 4 + hoff])
            dv = plsc.load_gather(adn_t, [dstv * 4 + hoff])
            tt = av + dv
            uu = jnp.exp(jnp.maximum(tt, 0.2 * tt) - cgv)
            ub[pl.ds(pl.multiple_of(g * 16, 8), 16)] = uu
            return 0
        lax.fori_loop(0, CHA // 4, group, 0)
        b4 = pl.multiple_of((base + ci * CHA) * 4, 8)
        pltpu.sync_copy(ub, u_hbm.at[pl.ds(b4, CHA * 4)])
        return 0
    lax.fori_loop(0, EPT // CHA, chunk, 0)


def _sc_gat_u(asn, adn, src, dst, cg):
    fn = pl.kernel(
        _sc_gat_u_body,
        out_type=jax.ShapeDtypeStruct((E * 4,), jnp.float32),
        mesh=_mesh(), compiler_params=CP,
        scratch_types=[
            pltpu.VMEM((N * 4,), jnp.float32),
            pltpu.VMEM((N * 4,), jnp.float32),
            pltpu.VMEM((16,), jnp.float32),
            pltpu.VMEM((CHA,), jnp.int32),
            pltpu.VMEM((CHA,), jnp.int32),
            pltpu.VMEM((CHA * 4,), jnp.float32),
        ],
    )
    return fn(asn, adn, src, dst, cg)


def _sc_dsum_body(u_hbm, dst_hbm, out_hbm, dstb, ub, rows, acc):
    c = lax.axis_index("c")
    s = lax.axis_index("s")
    base = (c * NS + s) * EPT

    _zero_buf(rows, CH, H)
    _zero_acc(s, rows, acc, CH)
    plsc.subcore_barrier()

    lane = jnp.arange(16, dtype=jnp.int32)
    lane4 = jnp.minimum(lane, 3)
    mask4 = (lane < 4).astype(jnp.float32)

    def chunk(ci, _):
        b = pl.multiple_of(base + ci * CH, 8)
        pltpu.sync_copy(dst_hbm.at[pl.ds(b, CH)], dstb)
        b4 = pl.multiple_of((base + ci * CH) * 4, 8)
        pltpu.sync_copy(u_hbm.at[pl.ds(b4, CH * 4)], ub)

        def edge(i, _):
            u16 = plsc.load_gather(ub, [i * 4 + lane4])
            rows[i, 0:16] = u16 * mask4
            return 0
        lax.fori_loop(0, CH, edge, 0)
        pltpu.sync_copy(rows, acc.at[dstb], add=True)
        return 0
    lax.fori_loop(0, NCHUNK, chunk, 0)

    plsc.subcore_barrier()
    _write_acc(c, s, acc, out_hbm, CH)


def _sc_dsum(u, dst):
    fn = pl.kernel(
        _sc_dsum_body,
        out_type=jax.ShapeDtypeStruct((NC, N, H), jnp.float32),
        mesh=_mesh(), compiler_params=CP,
        scratch_types=[
            pltpu.VMEM((CH,), jnp.int32),
            pltpu.VMEM((CH * 4,), jnp.float32),
            pltpu.VMEM((CH, H), jnp.float32),
            pltpu.VMEM_SHARED((N, H), jnp.float32),
        ],
    )
    return fn(u, dst)


CHA = 400


def _sc_att_body(u_hbm, dst_hbm, invd_hbm, att_hbm, invd_t, dstb, ub):
    c = lax.axis_index("c")
    s = lax.axis_index("s")
    base = (c * NS + s) * EPT

    pltpu.sync_copy(invd_hbm, invd_t)
    lane = jnp.arange(16, dtype=jnp.int32)
    eoff = lane // 4
    hoff = lane - eoff * 4

    def chunk(ci, _):
        b = pl.multiple_of(base + ci * CHA, 8)
        pltpu.sync_copy(dst_hbm.at[pl.ds(b, CHA)], dstb)
        b4 = pl.multiple_of((base + ci * CHA) * 4, 8)
        pltpu.sync_copy(u_hbm.at[pl.ds(b4, CHA * 4)], ub)

        def group(g, _):
            eidx = g * 4 + eoff
            dstv = plsc.load_gather(dstb, [eidx])
            iv = plsc.load_gather(invd_t, [dstv * 4 + hoff])
            off = pl.multiple_of(g * 16, 8)
            ub[pl.ds(off, 16)] = ub[pl.ds(off, 16)] * iv
            return 0
        lax.fori_loop(0, CHA // 4, group, 0)
        pltpu.sync_copy(ub, att_hbm.at[pl.ds(b4, CHA * 4)])
        return 0
    lax.fori_loop(0, EPT // CHA, chunk, 0)


def _sc_att(u, dst, invd):
    fn = pl.kernel(
        _sc_att_body,
        out_type=jax.ShapeDtypeStruct((E * 4,), jnp.float32),
        mesh=_mesh(), compiler_params=CP,
        scratch_types=[
            pltpu.VMEM((N * 4,), jnp.float32),
            pltpu.VMEM((CHA,), jnp.int32),
            pltpu.VMEM((CHA * 4,), jnp.float32),
        ],
    )
    return fn(u, dst, invd)


CHV = 40


def _sc_gat_value_body(xwg_hbm, src_hbm, dst_hbm, att_hbm, out_hbm,
                       srcb, dstb, attb, rows, combo, acc, sem):
    c = lax.axis_index("c")
    s = lax.axis_index("s")
    base = (c * NS + s) * EPT

    _zero_buf(combo, CHV, H)
    _zero_acc(s, combo, acc, CHV)
    plsc.subcore_barrier()

    def chunk(ci, _):
        b = pl.multiple_of(base + ci * CHV, 8)
        pltpu.sync_copy(src_hbm.at[pl.ds(b, CHV)], srcb)
        pltpu.sync_copy(dst_hbm.at[pl.ds(b, CHV)], dstb)
        b4 = pl.multiple_of((base + ci * CHV) * 4, 8)
        pltpu.sync_copy(att_hbm.at[pl.ds(b4, CHV * 4)], attb)
        pltpu.async_copy(xwg_hbm.at[srcb], rows, sem).wait()

        def edge(i, _):
            a = [plsc.load_gather(attb, [jnp.full((16,), i * 4 + h, jnp.int32)])
                 for h in range(HEADS)]
            for j in range(H // 16):
                acc16 = a[0] * rows[i, pl.ds(16 * j, 16)]
                for h in range(1, HEADS):
                    acc16 = acc16 + a[h] * rows[i, pl.ds(h * H + 16 * j, 16)]
                combo[i, pl.ds(16 * j, 16)] = acc16
            return 0
        lax.fori_loop(0, CHV, edge, 0)
        pltpu.sync_copy(combo, acc.at[dstb], add=True)
        return 0
    lax.fori_loop(0, EPT // CHV, chunk, 0)

    plsc.subcore_barrier()
    _write_acc(c, s, acc, out_hbm, CHV)


def _sc_gat_value(xwg, src, dst, att):
    fn = pl.kernel(
        _sc_gat_value_body,
        out_type=jax.ShapeDtypeStruct((NC, N, H), jnp.float32),
        mesh=_mesh(), compiler_params=CP,
        scratch_types=[
            pltpu.VMEM((CHV,), jnp.int32),
            pltpu.VMEM((CHV,), jnp.int32),
            pltpu.VMEM((CHV * 4,), jnp.float32),
            pltpu.VMEM((CHV, HEADS * H), jnp.float32),
            pltpu.VMEM((CHV, H), jnp.float32),
            pltpu.VMEM_SHARED((N, H), jnp.float32),
            pltpu.SemaphoreType.DMA,
        ],
    )
    return fn(xwg, src, dst, att)


def _mm_kernel(x_ref, w_ref, b_ref, o_ref, *, act):
    r = jnp.dot(x_ref[...], w_ref[...], preferred_element_type=jnp.float32)
    r = r + b_ref[...]
    o_ref[...] = act(r)


def _mm(x, w, b, act=lambda v: v):
    dout = w.shape[1]
    return pl.pallas_call(
        functools.partial(_mm_kernel, act=act),
        grid=(GRID,),
        in_specs=[
            pl.BlockSpec((BLK, x.shape[1]), lambda i: (i, 0)),
            pl.BlockSpec((w.shape[0], dout), lambda i: (0, 0)),
            pl.BlockSpec((1, dout), lambda i: (0, 0)),
        ],
        out_specs=pl.BlockSpec((BLK, dout), lambda i: (i, 0)),
        out_shape=jax.ShapeDtypeStruct((N, dout), jnp.float32),
    )(x, w, b.reshape(1, dout))


def _mm_scale_kernel(x_ref, w_ref, s_ref, o_ref):
    r = jnp.dot(x_ref[...], w_ref[...], preferred_element_type=jnp.float32)
    o_ref[...] = r * s_ref[...]


def _mm_scale(x, w, s):
    dout = w.shape[1]
    return pl.pallas_call(
        _mm_scale_kernel,
        grid=(GRID,),
        in_specs=[
            pl.BlockSpec((BLK, x.shape[1]), lambda i: (i, 0)),
            pl.BlockSpec((w.shape[0], dout), lambda i: (0, 0)),
            pl.BlockSpec((BLK, 1), lambda i: (i, 0)),
        ],
        out_specs=pl.BlockSpec((BLK, dout), lambda i: (i, 0)),
        out_shape=jax.ShapeDtypeStruct((N, dout), jnp.float32),
    )(x, w, s.reshape(N, 1))


def _gcn_post_kernel(a0_ref, a1_ref, y_ref, s_ref, b_ref, hp_ref, o_ref,
                     *, relu_res):
    r = (a0_ref[0] + a1_ref[0] + y_ref[...]) * s_ref[...] + b_ref[...]
    if relu_res:
        r = jnp.maximum(r, 0.0) + hp_ref[...]
    o_ref[...] = r


def _gcn_post(parts, y, dinv, b, h_prev, relu_res):
    return pl.pallas_call(
        functools.partial(_gcn_post_kernel, relu_res=relu_res),
        grid=(GRID,),
        in_specs=[
            pl.BlockSpec((1, BLK, H), lambda i: (0, i, 0)),
            pl.BlockSpec((1, BLK, H), lambda i: (1, i, 0)),
            pl.BlockSpec((BLK, H), lambda i: (i, 0)),
            pl.BlockSpec((BLK, 1), lambda i: (i, 0)),
            pl.BlockSpec((1, H), lambda i: (0, 0)),
            pl.BlockSpec((BLK, H), lambda i: (i, 0)),
        ],
        out_specs=pl.BlockSpec((BLK, H), lambda i: (i, 0)),
        out_shape=jax.ShapeDtypeStruct((N, H), jnp.float32),
    )(parts, parts, y, dinv.reshape(N, 1), b.reshape(1, H), h_prev)


def _gat_dense_kernel(x_ref, w_ref, as_ref, ad_ref, xw_ref, asn_ref, adn_ref):
    xw = jnp.dot(x_ref[...], w_ref[...], preferred_element_type=jnp.float32)
    xw_ref[...] = xw
    asn_ref[...] = jnp.dot(xw, as_ref[...], preferred_element_type=jnp.float32)
    adn_ref[...] = jnp.dot(xw, ad_ref[...], preferred_element_type=jnp.float32)


def _gat_dense(h3, W_gat, As, Ad):
    return pl.pallas_call(
        _gat_dense_kernel,
        grid=(GRID,),
        in_specs=[
            pl.BlockSpec((BLK, H), lambda i: (i, 0)),
            pl.BlockSpec((H, HEADS * H), lambda i: (0, 0)),
            pl.BlockSpec((HEADS * H, HEADS), lambda i: (0, 0)),
            pl.BlockSpec((HEADS * H, HEADS), lambda i: (0, 0)),
        ],
        out_specs=[
            pl.BlockSpec((BLK, HEADS * H), lambda i: (i, 0)),
            pl.BlockSpec((BLK, HEADS), lambda i: (i, 0)),
            pl.BlockSpec((BLK, HEADS), lambda i: (i, 0)),
        ],
        out_shape=[
            jax.ShapeDtypeStruct((N, HEADS * H), jnp.float32),
            jax.ShapeDtypeStruct((N, HEADS), jnp.float32),
            jax.ShapeDtypeStruct((N, HEADS), jnp.float32),
        ],
    )(h3, W_gat, As, Ad)


def _gat_mid_kernel(d0_ref, d1_ref, asn_ref, adn_ref, cg_ref, xw_ref,
                    invd_ref, st_ref):
    t = asn_ref[...] + adn_ref[...]
    u_self = jnp.exp(jnp.maximum(t, 0.2 * t) - cg_ref[...])
    den = d0_ref[0][:, :HEADS] + d1_ref[0][:, :HEADS] + u_self
    invd = 1.0 / den
    invd_ref[...] = invd
    w = u_self * invd
    xw = xw_ref[...]
    st = w[:, 0:1] * xw[:, 0:H]
    for h in range(1, HEADS):
        st = st + w[:, h:h + 1] * xw[:, h * H:(h + 1) * H]
    st_ref[...] = st


def _gat_mid(dparts, asn, adn, cg, xwg):
    return pl.pallas_call(
        _gat_mid_kernel,
        grid=(GRID,),
        in_specs=[
            pl.BlockSpec((1, BLK, H), lambda i: (0, i, 0)),
            pl.BlockSpec((1, BLK, H), lambda i: (1, i, 0)),
            pl.BlockSpec((BLK, HEADS), lambda i: (i, 0)),
            pl.BlockSpec((BLK, HEADS), lambda i: (i, 0)),
            pl.BlockSpec((1, HEADS), lambda i: (0, 0)),
            pl.BlockSpec((BLK, HEADS * H), lambda i: (i, 0)),
        ],
        out_specs=[
            pl.BlockSpec((BLK, HEADS), lambda i: (i, 0)),
            pl.BlockSpec((BLK, H), lambda i: (i, 0)),
        ],
        out_shape=[
            jax.ShapeDtypeStruct((N, HEADS), jnp.float32),
            jax.ShapeDtypeStruct((N, H), jnp.float32),
        ],
    )(dparts, dparts, asn, adn, cg.reshape(1, HEADS), xwg)


def _gat_final_kernel(h3_ref, p0_ref, p1_ref, st_ref, b_ref, wo_ref, bo_ref,
                      o_ref):
    h_att = ((p0_ref[0] + p1_ref[0] + st_ref[...]) * (1.0 / HEADS)
             + b_ref[...])
    hf = h3_ref[...] + h_att
    o_ref[...] = (jnp.dot(hf, wo_ref[...], preferred_element_type=jnp.float32)
                  + bo_ref[...])


def _gat_final(h3, vparts, st, b_gat, W_out, b_out):
    return pl.pallas_call(
        _gat_final_kernel,
        grid=(GRID,),
        in_specs=[
            pl.BlockSpec((BLK, H), lambda i: (i, 0)),
            pl.BlockSpec((1, BLK, H), lambda i: (0, i, 0)),
            pl.BlockSpec((1, BLK, H), lambda i: (1, i, 0)),
            pl.BlockSpec((BLK, H), lambda i: (i, 0)),
            pl.BlockSpec((1, H), lambda i: (0, 0)),
            pl.BlockSpec((H, 1), lambda i: (0, 0)),
            pl.BlockSpec((1, 1), lambda i: (0, 0)),
        ],
        out_specs=pl.BlockSpec((BLK, 1), lambda i: (i, 0)),
        out_shape=jax.ShapeDtypeStruct((N, 1), jnp.float32),
    )(h3, vparts, vparts, st, b_gat.reshape(1, H), W_out,
      b_out.reshape(1, 1))


def kernel(x, edge_index, edge_weight, W_in, b_in, W_g1, b_g1, W_g2, b_g2,
           W_g3, b_g3, W_gat, att_src, att_dst, b_gat, W_out, b_out):
    src = edge_index[0]
    dst = edge_index[1]
    ew = edge_weight

    ones = jnp.ones((N, H), jnp.float32)
    deg_parts = _sc_agg(ones, src, dst, ew)
    deg = deg_parts[0, :, 0] + deg_parts[1, :, 0] + 1.0
    dinv = lax.rsqrt(deg)

    h0 = _mm(x, W_in, b_in, act=lambda v: jnp.maximum(v, 0.0))

    def gcn(h, W, b, h_prev, relu_res):
        y = _mm_scale(h, W, dinv)
        parts = _sc_agg(y, src, dst, ew)
        return _gcn_post(parts, y, dinv, b, h_prev, relu_res)

    h1 = gcn(h0, W_g1, b_g1, h0, True)
    h2 = gcn(h1, W_g2, b_g2, h1, True)
    h3 = gcn(h2, W_g3, b_g3, h2, False)

    As = (jnp.eye(HEADS, dtype=jnp.float32)[:, None, :]
          * att_src[:, :, None]).reshape(HEADS * H, HEADS)
    Ad = (jnp.eye(HEADS, dtype=jnp.float32)[:, None, :]
          * att_dst[:, :, None]).reshape(HEADS * H, HEADS)
    xwg, asn, adn = _gat_dense(h3, W_gat, As, Ad)

    cg = jax.nn.leaky_relu(jnp.max(asn, axis=0) + jnp.max(adn, axis=0), 0.2)
    cg16 = jnp.tile(cg, 4)

    u = _sc_gat_u(asn.reshape(N * HEADS), adn.reshape(N * HEADS),
                  src, dst, cg16)
    dparts = _sc_dsum(u, dst)
    invd, st = _gat_mid(dparts, asn, adn, cg, xwg)
    att = _sc_att(u, dst, invd.reshape(N * HEADS))
    vparts = _sc_gat_value(xwg, src, dst, att)

    return _gat_final(h3, vparts, st, b_gat, W_out, b_out)
